# baseline (speedup 1.0000x reference)
import functools

import jax
import jax.numpy as jnp
from jax import lax
from jax.experimental import pallas as pl
from jax.experimental.pallas import tpu as pltpu
from jax.experimental.pallas import tpu_sc as plsc

H = 128
LANES = 16
NC, NS = 2, 16
NW = NC * NS
CH = 128
NSLOT = 4


@functools.partial(jax.jit, static_argnums=(5,))
def _run(table, idx2d, t_flat, w, b, n_rows):
    rows_w = n_rows // NW
    nchunk = rows_w // CH
    mesh = plsc.VectorSubcoreMesh(core_axis_name="c", subcore_axis_name="s")

    @functools.partial(
        pl.kernel,
        mesh=mesh,
        out_type=jax.ShapeDtypeStruct((n_rows, H), jnp.float32),
        scratch_types=[
            pltpu.VMEM((rows_w // 128, 128), jnp.int32),
            pltpu.VMEM((rows_w,), jnp.float32),
            pltpu.VMEM((NSLOT, CH, H), jnp.float32),
            pltpu.VMEM((H,), jnp.float32),
            pltpu.VMEM((H,), jnp.float32),
        ] + [pltpu.SemaphoreType.DMA] * (2 * NSLOT),
    )
    def k(table_hbm, idx_hbm, t_hbm, w_hbm, b_hbm, out_hbm,
          idx_v, t_v, rows_v, w_v, b_v, *sems):
        sg = sems[:NSLOT]
        so = sems[NSLOT:]
        wid = lax.axis_index("s") * NC + lax.axis_index("c")
        row0 = wid * rows_w
        pltpu.sync_copy(w_hbm, w_v)
        pltpu.sync_copy(b_hbm, b_v)
        pltpu.sync_copy(idx_hbm.at[pl.ds(wid * (rows_w // 128),
                                         rows_w // 128)], idx_v)
        pltpu.sync_copy(t_hbm.at[pl.ds(row0, rows_w)], t_v)
        wj = [w_v[pl.ds(LANES * j, LANES)] for j in range(H // LANES)]
        bj = [b_v[pl.ds(LANES * j, LANES)] for j in range(H // LANES)]

        def gather_copy(c, slot):
            return pltpu.make_async_copy(
                table_hbm.at[idx_v.at[c]], rows_v.at[slot], sg[slot])

        def out_copy(c, slot):
            return pltpu.make_async_copy(
                rows_v.at[slot], out_hbm.at[pl.ds(row0 + c * CH, CH)],
                so[slot])

        def compute(c, slot):
            def grp(g, carry):
                tv16 = t_v[pl.ds(c * CH + g * LANES, LANES)]
                for r in range(LANES):
                    tb = lax.broadcast(tv16[r], (LANES,))
                    i = g * LANES + r
                    for j in range(H // LANES):
                        plsc.addupdate(
                            rows_v.at[slot, i, pl.ds(LANES * j, LANES)],
                            tb * wj[j] + bj[j])
                return carry

            lax.fori_loop(0, CH // LANES, grp, 0)

        def body(c, slot, first=False, last=False):
            ns = (slot + 2) % NSLOT
            if not first:
                out_copy(c - 2, ns).wait()
            if not last:
                gather_copy(c + 2, ns).start()
            gather_copy(c, slot).wait()
            compute(c, slot)
            out_copy(c, slot).start()

        gather_copy(0, 0).start()
        gather_copy(1, 1).start()
        body(0, 0, first=True)
        body(1, 1, first=True)

        def outer(it, carry):
            cbase = 2 + 4 * it
            for kk in range(4):
                body(cbase + kk, (2 + kk) % NSLOT)
            return carry

        lax.fori_loop(0, (nchunk - 4) // 4, outer, 0)

        body(nchunk - 2, (nchunk - 2) % NSLOT, last=True)
        body(nchunk - 1, (nchunk - 1) % NSLOT, last=True)
        out_copy(nchunk - 2, (nchunk - 2) % NSLOT).wait()
        out_copy(nchunk - 1, (nchunk - 1) % NSLOT).wait()

    return k(table, idx2d, t_flat, w, b)


def kernel(seq_t, seq_types, type_table, Wt_w, Wt_b):
    bsz, seq_len = seq_t.shape
    n_rows = bsz * seq_len
    idx2d = seq_types.astype(jnp.int32).reshape(n_rows // 128, 128)
    t_flat = seq_t.reshape(n_rows)
    w = Wt_w.reshape(H)
    out = _run(type_table, idx2d, t_flat, w, Wt_b, n_rows)
    return out.reshape(bsz, seq_len, H)
